# baseline (speedup 1.0000x reference)
import functools

import jax
import jax.numpy as jnp
from jax import lax
from jax.experimental import pallas as pl
from jax.experimental.pallas import tpu as pltpu
from jax.experimental.pallas import tpu_sc as plsc

L = 16
K = 8
CHUNK_ROWS = 16


def _sort_dir(x, ascending):
    if ascending:
        return lax.sort(x)
    k, _ = plsc.sort_key_val(x, x, descending=True)
    return k


def _row_thresholds(ib, r, n_cols):
    nvec = n_cols // L
    lane = lax.iota(jnp.int32, L)

    leaves = [
        _sort_dir(ib[r, pl.ds(L * i, L)], ascending=(i % 2 == 0))
        for i in range(nvec)
    ]

    tops = leaves
    bots = leaves
    while len(tops) > 1:
        new_tops = []
        new_bots = []
        for j in range(len(tops) // 2):
            asc = j % 2 == 0
            new_tops.append(_sort_dir(jnp.maximum(tops[2 * j], tops[2 * j + 1]), asc))
            new_bots.append(_sort_dir(jnp.minimum(bots[2 * j], bots[2 * j + 1]), asc))
        tops = new_tops
        bots = new_bots

    tp = jnp.min(jnp.where(lane >= L - K, tops[0], jnp.inf))
    tn = jnp.max(jnp.where(lane < K, bots[0], -jnp.inf))
    return jnp.broadcast_to(tp, (L,)), jnp.broadcast_to(tn, (L,))


def _mask_pass(ib, ob, r, tp, tn, n_cols):
    one = jnp.full((L,), 1.0, jnp.float32)
    zero = jnp.zeros((L,), jnp.float32)
    neg_one = jnp.full((L,), -1.0, jnp.float32)
    for i in range(n_cols // L):
        x = ib[r, pl.ds(L * i, L)]
        y = jnp.where(x >= tp, one, zero)
        y = jnp.where(x <= tn, neg_one, y)
        ob[r, pl.ds(L * i, L)] = y


def _build(rows, n_cols):
    info = plsc.get_sparse_core_info()
    num_workers = info.num_cores * info.num_subcores
    rows_per_worker = rows // num_workers
    n_chunks = rows_per_worker // CHUNK_ROWS
    n_groups = n_chunks // 2
    mesh = plsc.VectorSubcoreMesh(core_axis_name="c", subcore_axis_name="s")
    buf_t = pltpu.VMEM((CHUNK_ROWS, n_cols), jnp.float32)

    @functools.partial(
        pl.kernel,
        out_type=jax.ShapeDtypeStruct((rows, n_cols), jnp.float32),
        mesh=mesh,
        scratch_types=[buf_t, buf_t, buf_t, buf_t,
                       pltpu.SemaphoreType.DMA, pltpu.SemaphoreType.DMA,
                       pltpu.SemaphoreType.DMA, pltpu.SemaphoreType.DMA],
        compiler_params=pltpu.CompilerParams(needs_layout_passes=False),
    )
    def mask_kernel(x_hbm, out_hbm, ib0, ib1, ob0, ob1,
                    si0, si1, so0, so1):
        wid = lax.axis_index("s") * info.num_cores + lax.axis_index("c")
        base = wid * rows_per_worker

        def in_slice(c):
            return x_hbm.at[pl.ds(base + c * CHUNK_ROWS, CHUNK_ROWS)]

        def out_slice(c):
            return out_hbm.at[pl.ds(base + c * CHUNK_ROWS, CHUNK_ROWS)]

        def compute(ib, ob):
            first = _row_thresholds(ib, 0, n_cols)

            def row_body(h, carry):
                tp_prev, tn_prev = carry
                _mask_pass(ib, ob, h - 1, tp_prev, tn_prev, n_cols)
                return _row_thresholds(ib, h, n_cols)

            tp_last, tn_last = lax.fori_loop(1, CHUNK_ROWS, row_body, first)
            _mask_pass(ib, ob, CHUNK_ROWS - 1, tp_last, tn_last, n_cols)

        pltpu.async_copy(in_slice(0), ib0, si0)
        pltpu.async_copy(in_slice(1), ib1, si1)

        def step(c, ib, ob, si, so, wait_out, start_in):
            pltpu.make_async_copy(in_slice(c), ib, si).wait()
            if wait_out:
                pltpu.make_async_copy(ob, out_slice(c), so).wait()
            compute(ib, ob)
            pltpu.async_copy(ob, out_slice(c), so)
            if start_in:
                pltpu.async_copy(in_slice(c + 2), ib, si)

        step(0, ib0, ob0, si0, so0, wait_out=False, start_in=True)
        step(1, ib1, ob1, si1, so1, wait_out=False, start_in=True)

        def group_body(g, carry):
            c = 2 * g
            step(c, ib0, ob0, si0, so0, wait_out=True, start_in=True)
            step(c + 1, ib1, ob1, si1, so1, wait_out=True, start_in=True)
            return carry

        lax.fori_loop(1, n_groups - 1, group_body, 0)

        step(n_chunks - 2, ib0, ob0, si0, so0, wait_out=True, start_in=False)
        step(n_chunks - 1, ib1, ob1, si1, so1, wait_out=True, start_in=False)

        pltpu.make_async_copy(ob0, out_slice(n_chunks - 2), so0).wait()
        pltpu.make_async_copy(ob1, out_slice(n_chunks - 1), so1).wait()

    return mask_kernel


def kernel(tensor):
    b, n, m = tensor.shape
    x = tensor.reshape(b * n, m)
    out = _build(b * n, m)(x)
    return out.reshape(b, n, m)


# speedup vs baseline: 1.0701x; 1.0701x over previous
import functools

import jax
import jax.numpy as jnp
from jax import lax
from jax.experimental import pallas as pl
from jax.experimental.pallas import tpu as pltpu
from jax.experimental.pallas import tpu_sc as plsc

L = 16
K = 8
CHUNK_ROWS = 16


def _sort_dir(x, ascending):
    if ascending:
        return lax.sort(x)
    k, _ = plsc.sort_key_val(x, x, descending=True)
    return k


def _row_mask(ib, ob, r, n_cols):
    nvec = n_cols // L
    lane = lax.iota(jnp.int32, L)

    leaves = [
        _sort_dir(ib[r, pl.ds(L * i, L)], ascending=(i % 2 == 0))
        for i in range(nvec)
    ]

    def merge_level(nodes, op):
        return [
            _sort_dir(op(nodes[2 * j], nodes[2 * j + 1]), ascending=(j % 2 == 0))
            for j in range(len(nodes) // 2)
        ]

    tops = leaves
    bots = leaves
    while len(tops) > 1:
        bots = merge_level(bots, jnp.minimum)
        tops = merge_level(tops, jnp.maximum)

    idx_tp = jnp.full((L,), L - K, jnp.int32)
    idx_tn = jnp.full((L,), K - 1, jnp.int32)
    tp = tops[0].at[idx_tp].get(mode="promise_in_bounds")
    tn = bots[0].at[idx_tn].get(mode="promise_in_bounds")

    one = jnp.full((L,), 1.0, jnp.float32)
    zero = jnp.zeros((L,), jnp.float32)
    neg_one = jnp.full((L,), -1.0, jnp.float32)
    for i in range(nvec):
        x = ib[r, pl.ds(L * i, L)]
        y = jnp.where(x >= tp, one, zero)
        y = jnp.where(x <= tn, neg_one, y)
        ob[r, pl.ds(L * i, L)] = y


def _build(rows, n_cols):
    info = plsc.get_sparse_core_info()
    num_workers = info.num_cores * info.num_subcores
    rows_per_worker = rows // num_workers
    n_chunks = rows_per_worker // CHUNK_ROWS
    n_groups = n_chunks // 2
    mesh = plsc.VectorSubcoreMesh(core_axis_name="c", subcore_axis_name="s")
    buf_t = pltpu.VMEM((CHUNK_ROWS, n_cols), jnp.float32)

    @functools.partial(
        pl.kernel,
        out_type=jax.ShapeDtypeStruct((rows, n_cols), jnp.float32),
        mesh=mesh,
        scratch_types=[buf_t, buf_t, buf_t, buf_t,
                       pltpu.SemaphoreType.DMA, pltpu.SemaphoreType.DMA,
                       pltpu.SemaphoreType.DMA, pltpu.SemaphoreType.DMA],
        compiler_params=pltpu.CompilerParams(needs_layout_passes=False),
    )
    def mask_kernel(x_hbm, out_hbm, ib0, ib1, ob0, ob1,
                    si0, si1, so0, so1):
        wid = lax.axis_index("s") * info.num_cores + lax.axis_index("c")
        base = wid * rows_per_worker

        def in_slice(c):
            return x_hbm.at[pl.ds(base + c * CHUNK_ROWS, CHUNK_ROWS)]

        def out_slice(c):
            return out_hbm.at[pl.ds(base + c * CHUNK_ROWS, CHUNK_ROWS)]

        def compute(ib, ob):
            def row_body(r, rc):
                _row_mask(ib, ob, r, n_cols)
                return rc
            lax.fori_loop(0, CHUNK_ROWS, row_body, 0)

        pltpu.async_copy(in_slice(0), ib0, si0)
        pltpu.async_copy(in_slice(1), ib1, si1)

        def step(c, ib, ob, si, so, wait_out, start_in):
            pltpu.make_async_copy(in_slice(c), ib, si).wait()
            if wait_out:
                pltpu.make_async_copy(ob, out_slice(c), so).wait()
            compute(ib, ob)
            pltpu.async_copy(ob, out_slice(c), so)
            if start_in:
                pltpu.async_copy(in_slice(c + 2), ib, si)

        step(0, ib0, ob0, si0, so0, wait_out=False, start_in=True)
        step(1, ib1, ob1, si1, so1, wait_out=False, start_in=True)

        def group_body(g, carry):
            c = 2 * g
            step(c, ib0, ob0, si0, so0, wait_out=True, start_in=True)
            step(c + 1, ib1, ob1, si1, so1, wait_out=True, start_in=True)
            return carry

        lax.fori_loop(1, n_groups - 1, group_body, 0)

        step(n_chunks - 2, ib0, ob0, si0, so0, wait_out=True, start_in=False)
        step(n_chunks - 1, ib1, ob1, si1, so1, wait_out=True, start_in=False)

        pltpu.make_async_copy(ob0, out_slice(n_chunks - 2), so0).wait()
        pltpu.make_async_copy(ob1, out_slice(n_chunks - 1), so1).wait()

    return mask_kernel


def kernel(tensor):
    b, n, m = tensor.shape
    x = tensor.reshape(b * n, m)
    out = _build(b * n, m)(x)
    return out.reshape(b, n, m)
